# baseline (device time: 71890 ns/iter reference)
import jax
import jax.numpy as jnp
from jax import lax
from jax.experimental import pallas as pl
from jax.experimental.pallas import tpu as pltpu

S = 8


def kernel(x):
    _, m, n_tot = x.shape
    n_half = n_tot // 2
    q_rows = m // 4
    sub = q_rows // S
    MESH = pl.DeviceIdType.MESH

    def body(
        x_ref, out_ref,
        stage_a, stage_b, recv_za, recv_zb, recv_x, recv_y,
        za_ssem, zb_ssem, fx_ssem, fy_ssem,
        za_rsem, zb_rsem, fx_rsem, fy_rsem,
    ):
        my_x = lax.axis_index("x")
        my_y = lax.axis_index("y")
        my_z = lax.axis_index("z")
        other_z = 1 - my_z
        partner = (my_x, my_y, other_z)
        xn = (1 - my_x, my_y, my_z)
        yn = (my_x, 1 - my_y, my_z)
        q = 2 * my_x + my_y
        qx = 2 * (1 - my_x) + my_y
        qy = 2 * my_x + (1 - my_y)
        qd = 2 * (1 - my_x) + (1 - my_y)

        barrier_sem = pltpu.get_barrier_semaphore()
        for nbr in (partner, xn, yn):
            pl.semaphore_signal(
                barrier_sem, inc=1, device_id=nbr, device_id_type=MESH
            )
        pl.semaphore_wait(barrier_sem, 3)

        oc = pl.ds(other_z * n_half, n_half)
        zc = pl.ds(my_z * n_half, n_half)

        def add_sub(r_idx, s, buf):
            rows = pl.ds(r_idx * q_rows + s * sub, sub)
            out_ref[rows, :] = (
                x_ref[0, rows, zc]
                + buf[pl.ds(s * sub, sub), :].astype(jnp.float32)
            ).astype(jnp.bfloat16)

        za_rdma, zb_rdma = [], []
        for s in range(S):
            stage_a[pl.ds(s * sub, sub), :] = x_ref[
                0, pl.ds(q * q_rows + s * sub, sub), oc
            ].astype(jnp.bfloat16)
            r = pltpu.make_async_remote_copy(
                src_ref=stage_a.at[pl.ds(s * sub, sub)],
                dst_ref=recv_za.at[pl.ds(s * sub, sub)],
                send_sem=za_ssem.at[s],
                recv_sem=za_rsem.at[s],
                device_id=partner,
                device_id_type=MESH,
            )
            r.start()
            za_rdma.append(r)
        for s in range(S):
            stage_b[pl.ds(s * sub, sub), :] = x_ref[
                0, pl.ds(qd * q_rows + s * sub, sub), oc
            ].astype(jnp.bfloat16)
            r = pltpu.make_async_remote_copy(
                src_ref=stage_b.at[pl.ds(s * sub, sub)],
                dst_ref=recv_zb.at[pl.ds(s * sub, sub)],
                send_sem=zb_ssem.at[s],
                recv_sem=zb_rsem.at[s],
                device_id=partner,
                device_id_type=MESH,
            )
            r.start()
            zb_rdma.append(r)

        fx_rdma, fy_rdma = [], []
        for s in range(S):
            za_rdma[s].wait_recv()
            rx = pltpu.make_async_remote_copy(
                src_ref=recv_za.at[pl.ds(s * sub, sub)],
                dst_ref=recv_x.at[pl.ds(s * sub, sub)],
                send_sem=fx_ssem.at[s],
                recv_sem=fx_rsem.at[s],
                device_id=xn,
                device_id_type=MESH,
            )
            rx.start()
            fx_rdma.append(rx)
            ry = pltpu.make_async_remote_copy(
                src_ref=recv_za.at[pl.ds(s * sub, sub)],
                dst_ref=recv_y.at[pl.ds(s * sub, sub)],
                send_sem=fy_ssem.at[s],
                recv_sem=fy_rsem.at[s],
                device_id=yn,
                device_id_type=MESH,
            )
            ry.start()
            fy_rdma.append(ry)
            add_sub(q, s, recv_za)

        for s in range(S):
            fx_rdma[s].wait_recv()
            add_sub(qx, s, recv_x)
            fy_rdma[s].wait_recv()
            add_sub(qy, s, recv_y)

        for s in range(S):
            zb_rdma[s].wait_recv()
            add_sub(qd, s, recv_zb)

        for s in range(S):
            za_rdma[s].wait_send()
            zb_rdma[s].wait_send()
            fx_rdma[s].wait_send()
            fy_rdma[s].wait_send()

    return pl.pallas_call(
        body,
        out_shape=jax.ShapeDtypeStruct((m, n_half), jnp.bfloat16),
        in_specs=[pl.BlockSpec(memory_space=pltpu.VMEM)],
        out_specs=pl.BlockSpec(memory_space=pltpu.VMEM),
        scratch_shapes=[
            pltpu.VMEM((q_rows, n_half), jnp.bfloat16),
            pltpu.VMEM((q_rows, n_half), jnp.bfloat16),
            pltpu.VMEM((q_rows, n_half), jnp.bfloat16),
            pltpu.VMEM((q_rows, n_half), jnp.bfloat16),
            pltpu.VMEM((q_rows, n_half), jnp.bfloat16),
            pltpu.VMEM((q_rows, n_half), jnp.bfloat16),
            pltpu.SemaphoreType.DMA((S,)),
            pltpu.SemaphoreType.DMA((S,)),
            pltpu.SemaphoreType.DMA((S,)),
            pltpu.SemaphoreType.DMA((S,)),
            pltpu.SemaphoreType.DMA((S,)),
            pltpu.SemaphoreType.DMA((S,)),
            pltpu.SemaphoreType.DMA((S,)),
            pltpu.SemaphoreType.DMA((S,)),
        ],
        compiler_params=pltpu.CompilerParams(
            collective_id=0, vmem_limit_bytes=100 * 1024 * 1024
        ),
    )(x)


# device time: 52498 ns/iter; 1.3694x vs baseline; 1.3694x over previous
import jax
import jax.numpy as jnp
from jax import lax
from jax.experimental import pallas as pl
from jax.experimental.pallas import tpu as pltpu

S = 16
ZD = 6
XD_LO, XD_HI = 6, 11


def kernel(x):
    _, m, n_tot = x.shape
    n_half = n_tot // 2
    q_rows = m // 4
    sub = q_rows // S
    MESH = pl.DeviceIdType.MESH

    def body(
        x_hbm, out_hbm,
        xq_f32, xd_f32, xloc, stage_a, stage_d,
        recv_q, recv_x, recv_y, recv_d, out_v,
        qdma_sem, ddma_sem, xloc_sem, odma_sem,
        za_ssem, zd_ssem, fx_ssem, fy_ssem, dx_ssem, dy_ssem,
        za_rsem, zd_rsem, fx_rsem, fy_rsem, dx_rsem, dy_rsem,
    ):
        my_x = lax.axis_index("x")
        my_y = lax.axis_index("y")
        my_z = lax.axis_index("z")
        other_z = 1 - my_z
        partner = (my_x, my_y, other_z)
        xn = (1 - my_x, my_y, my_z)
        yn = (my_x, 1 - my_y, my_z)
        q = 2 * my_x + my_y
        qx = 2 * (1 - my_x) + my_y
        qy = 2 * my_x + (1 - my_y)
        qd = 2 * (1 - my_x) + (1 - my_y)

        barrier_sem = pltpu.get_barrier_semaphore()
        for nbr in (partner, xn, yn):
            pl.semaphore_signal(
                barrier_sem, inc=1, device_id=nbr, device_id_type=MESH
            )
        pl.semaphore_wait(barrier_sem, 3)

        oc = pl.ds(other_z * n_half, n_half)
        zc = pl.ds(my_z * n_half, n_half)

        xloc_dma = []
        for k, r_idx in enumerate((q, qy, qx, qd)):
            c = pltpu.make_async_copy(
                x_hbm.at[0, pl.ds(r_idx * q_rows, q_rows), zc],
                xloc.at[k],
                xloc_sem.at[k],
            )
            c.start()
            xloc_dma.append(c)

        qdma = []
        for s in range(S):
            c = pltpu.make_async_copy(
                x_hbm.at[0, pl.ds(q * q_rows + s * sub, sub), oc],
                xq_f32.at[pl.ds(s * sub, sub)],
                qdma_sem.at[s],
            )
            c.start()
            qdma.append(c)
        ddma = []
        for j in range(ZD):
            c = pltpu.make_async_copy(
                x_hbm.at[0, pl.ds(qd * q_rows + j * sub, sub), oc],
                xd_f32.at[pl.ds(j * sub, sub)],
                ddma_sem.at[j],
            )
            c.start()
            ddma.append(c)

        za_rdma = []
        for s in range(S):
            qdma[s].wait()
            stage_a[pl.ds(s * sub, sub), :] = xq_f32[
                pl.ds(s * sub, sub), :
            ].astype(jnp.bfloat16)
            r = pltpu.make_async_remote_copy(
                src_ref=stage_a.at[pl.ds(s * sub, sub)],
                dst_ref=recv_q.at[pl.ds(s * sub, sub)],
                send_sem=za_ssem.at[s],
                recv_sem=za_rsem.at[s],
                device_id=partner,
                device_id_type=MESH,
            )
            r.start()
            za_rdma.append(r)
        zd_rdma = []
        for j in range(ZD):
            ddma[j].wait()
            stage_d[pl.ds(j * sub, sub), :] = xd_f32[
                pl.ds(j * sub, sub), :
            ].astype(jnp.bfloat16)
            r = pltpu.make_async_remote_copy(
                src_ref=stage_d.at[pl.ds(j * sub, sub)],
                dst_ref=recv_d.at[pl.ds(j * sub, sub)],
                send_sem=zd_ssem.at[j],
                recv_sem=zd_rsem.at[j],
                device_id=partner,
                device_id_type=MESH,
            )
            r.start()
            zd_rdma.append(r)

        def add_sub(r_idx, k, s, buf):
            rows = pl.ds(s * sub, sub)
            out_v[pl.ds(r_idx * q_rows + s * sub, sub), :] = (
                xloc[k, pl.ds(s * sub, sub), :]
                + buf[rows, :].astype(jnp.float32)
            ).astype(jnp.bfloat16)

        def store_quarter(r_idx, k):
            rows = pl.ds(r_idx * q_rows, q_rows)
            c = pltpu.make_async_copy(
                out_v.at[rows], out_hbm.at[rows], odma_sem.at[k]
            )
            c.start()
            return c

        fx_rdma, fy_rdma = [], []
        xloc_dma[0].wait()
        for s in range(S):
            za_rdma[s].wait_recv()
            rx = pltpu.make_async_remote_copy(
                src_ref=recv_q.at[pl.ds(s * sub, sub)],
                dst_ref=recv_x.at[pl.ds(s * sub, sub)],
                send_sem=fx_ssem.at[s],
                recv_sem=fx_rsem.at[s],
                device_id=xn,
                device_id_type=MESH,
            )
            rx.start()
            fx_rdma.append(rx)
            ry = pltpu.make_async_remote_copy(
                src_ref=recv_q.at[pl.ds(s * sub, sub)],
                dst_ref=recv_y.at[pl.ds(s * sub, sub)],
                send_sem=fy_ssem.at[s],
                recv_sem=fy_rsem.at[s],
                device_id=yn,
                device_id_type=MESH,
            )
            ry.start()
            fy_rdma.append(ry)
            add_sub(q, 0, s, recv_q)
        out_q = store_quarter(q, 0)

        dx_rdma, dy_rdma = {}, {}
        for s in range(XD_LO, XD_HI):
            fy_rdma[s].wait_recv()
            rd = pltpu.make_async_remote_copy(
                src_ref=recv_y.at[pl.ds(s * sub, sub)],
                dst_ref=recv_d.at[pl.ds(s * sub, sub)],
                send_sem=dx_ssem.at[s - XD_LO],
                recv_sem=dx_rsem.at[s - XD_LO],
                device_id=xn,
                device_id_type=MESH,
            )
            rd.start()
            dx_rdma[s] = rd
        for s in range(XD_HI, S):
            fx_rdma[s].wait_recv()
            rd = pltpu.make_async_remote_copy(
                src_ref=recv_x.at[pl.ds(s * sub, sub)],
                dst_ref=recv_d.at[pl.ds(s * sub, sub)],
                send_sem=dy_ssem.at[s - XD_HI],
                recv_sem=dy_rsem.at[s - XD_HI],
                device_id=yn,
                device_id_type=MESH,
            )
            rd.start()
            dy_rdma[s] = rd

        xloc_dma[1].wait()
        for s in list(range(XD_LO)) + list(range(XD_HI, S)):
            fy_rdma[s].wait_recv()
        for s in range(S):
            add_sub(qy, 1, s, recv_y)
        out_qy = store_quarter(qy, 1)
        xloc_dma[2].wait()
        for s in range(XD_HI):
            fx_rdma[s].wait_recv()
        for s in range(S):
            add_sub(qx, 2, s, recv_x)
        out_qx = store_quarter(qx, 2)

        xloc_dma[3].wait()
        for j in range(ZD):
            zd_rdma[j].wait_recv()
            add_sub(qd, 3, j, recv_d)
        for s in range(XD_LO, XD_HI):
            dx_rdma[s].wait_recv()
            add_sub(qd, 3, s, recv_d)
        for s in range(XD_HI, S):
            dy_rdma[s].wait_recv()
            add_sub(qd, 3, s, recv_d)
        out_qd = store_quarter(qd, 3)

        for s in range(S):
            za_rdma[s].wait_send()
            fx_rdma[s].wait_send()
            fy_rdma[s].wait_send()
        for j in range(ZD):
            zd_rdma[j].wait_send()
        for s in range(XD_LO, XD_HI):
            dx_rdma[s].wait_send()
        for s in range(XD_HI, S):
            dy_rdma[s].wait_send()
        for c in (out_q, out_qy, out_qx, out_qd):
            c.wait()

    return pl.pallas_call(
        body,
        out_shape=jax.ShapeDtypeStruct((m, n_half), jnp.bfloat16),
        in_specs=[pl.BlockSpec(memory_space=pl.ANY)],
        out_specs=pl.BlockSpec(memory_space=pl.ANY),
        scratch_shapes=[
            pltpu.VMEM((q_rows, n_half), jnp.float32),
            pltpu.VMEM((ZD * sub, n_half), jnp.float32),
            pltpu.VMEM((4, q_rows, n_half), jnp.float32),
            pltpu.VMEM((q_rows, n_half), jnp.bfloat16),
            pltpu.VMEM((ZD * sub, n_half), jnp.bfloat16),
            pltpu.VMEM((q_rows, n_half), jnp.bfloat16),
            pltpu.VMEM((q_rows, n_half), jnp.bfloat16),
            pltpu.VMEM((q_rows, n_half), jnp.bfloat16),
            pltpu.VMEM((q_rows, n_half), jnp.bfloat16),
            pltpu.VMEM((m, n_half), jnp.bfloat16),
            pltpu.SemaphoreType.DMA((S,)),
            pltpu.SemaphoreType.DMA((ZD,)),
            pltpu.SemaphoreType.DMA((4,)),
            pltpu.SemaphoreType.DMA((4,)),
            pltpu.SemaphoreType.DMA((S,)),
            pltpu.SemaphoreType.DMA((ZD,)),
            pltpu.SemaphoreType.DMA((S,)),
            pltpu.SemaphoreType.DMA((S,)),
            pltpu.SemaphoreType.DMA((XD_HI - XD_LO,)),
            pltpu.SemaphoreType.DMA((S - XD_HI,)),
            pltpu.SemaphoreType.DMA((S,)),
            pltpu.SemaphoreType.DMA((ZD,)),
            pltpu.SemaphoreType.DMA((S,)),
            pltpu.SemaphoreType.DMA((S,)),
            pltpu.SemaphoreType.DMA((XD_HI - XD_LO,)),
            pltpu.SemaphoreType.DMA((S - XD_HI,)),
        ],
        compiler_params=pltpu.CompilerParams(
            collective_id=0, vmem_limit_bytes=100 * 1024 * 1024
        ),
    )(x)
